# Initial kernel scaffold; baseline (speedup 1.0000x reference)
import functools

import jax
import jax.numpy as jnp
from jax import lax
from jax.experimental import pallas as pl
from jax.experimental.pallas import tpu as pltpu
from jax.experimental.pallas import tpu_sc as plsc

_NC = 2
_NS = 16
_CHUNK = 128


def _sc_mesh():
    return plsc.VectorSubcoreMesh(
        core_axis_name="c", subcore_axis_name="s", num_cores=_NC, num_subcores=_NS
    )


def _deg_body(dst_hbm, zeros_hbm, ones_hbm, out_hbm, dstv, onesv, deg_sh):
    c = lax.axis_index("c")
    s = lax.axis_index("s")
    tid = c * _NS + s
    nchunk = dstv.shape[0]
    rpt = deg_sh.shape[0] // _NS
    pltpu.sync_copy(dst_hbm.at[pl.ds(tid * nchunk, nchunk)], dstv)
    pltpu.sync_copy(ones_hbm, onesv)
    pltpu.sync_copy(zeros_hbm, deg_sh.at[pl.ds(s * rpt, rpt)])
    plsc.subcore_barrier()

    def body(g, carry):
        pltpu.sync_copy(onesv, deg_sh.at[dstv.at[g]], add=True)
        return carry

    lax.fori_loop(0, nchunk, body, 0)
    plsc.subcore_barrier()
    pltpu.sync_copy(deg_sh.at[pl.ds(s * rpt, rpt)], out_hbm.at[pl.ds(tid * rpt, rpt)])


def _gather_scatter_body(xwt_hbm, src_hbm, dst_hbm, zeros_hbm, out_hbm,
                         srcv, dstv, buf, acc_sh):
    c = lax.axis_index("c")
    s = lax.axis_index("s")
    tid = c * _NS + s
    nchunk = srcv.shape[0]
    rpt = acc_sh.shape[0] // _NS
    pltpu.sync_copy(src_hbm.at[pl.ds(tid * nchunk, nchunk)], srcv)
    pltpu.sync_copy(dst_hbm.at[pl.ds(tid * nchunk, nchunk)], dstv)
    pltpu.sync_copy(zeros_hbm, acc_sh.at[pl.ds(s * rpt, rpt)])
    plsc.subcore_barrier()

    def body(g, carry):
        pltpu.sync_copy(xwt_hbm.at[srcv.at[g]], buf)
        pltpu.sync_copy(buf, acc_sh.at[dstv.at[g]], add=True)
        return carry

    lax.fori_loop(0, nchunk, body, 0)
    plsc.subcore_barrier()
    pltpu.sync_copy(acc_sh.at[pl.ds(s * rpt, rpt)], out_hbm.at[pl.ds(tid * rpt, rpt)])


def _dinv_block(deg_ref, row0, bm, n_valid, npad):
    row = row0 + lax.broadcasted_iota(jnp.int32, (bm, 1), 0)
    valid = (row % npad) < n_valid
    return jnp.where(valid, lax.rsqrt(deg_ref[...] + 1.0), 0.0)


def _mm_scale_kernel(deg_ref, x_ref, w_ref, o_ref, *, n_valid, npad, bm):
    dinv = _dinv_block(deg_ref, pl.program_id(0) * bm, bm, n_valid, npad)
    o_ref[...] = dinv * jnp.dot(x_ref[...], w_ref[...],
                                preferred_element_type=jnp.float32)


def _mm_mid_kernel(deg_ref, part_ref, xwt_ref, b_ref, w_ref, o_ref,
                   *, n_valid, npad, bm):
    dinv = _dinv_block(deg_ref, pl.program_id(0) * bm, bm, n_valid, npad)
    h = jnp.maximum(dinv * (part_ref[...] + xwt_ref[...]) + b_ref[...], 0.0)
    o_ref[...] = dinv * jnp.dot(h, w_ref[...], preferred_element_type=jnp.float32)


def _pool_kernel(deg_ref, part_ref, xwt_ref, b_ref, batch_ref,
                 pooled_ref, cnt_ref, *, n_valid, npad, bk, g):
    k = pl.program_id(1)
    dinv = _dinv_block(deg_ref, k * bk, bk, n_valid, npad)
    o3 = dinv * (part_ref[...] + xwt_ref[...]) + b_ref[...]
    batv = batch_ref[0, :]
    oneh = (lax.broadcasted_iota(jnp.int32, (g, bk), 0) == batv[None, :]).astype(
        jnp.float32)

    @pl.when(k == 0)
    def _():
        pooled_ref[...] = jnp.zeros_like(pooled_ref)
        cnt_ref[...] = jnp.zeros_like(cnt_ref)

    pooled_ref[...] += jnp.dot(oneh, o3, preferred_element_type=jnp.float32)[None]
    cnt_ref[...] += jnp.broadcast_to(
        jnp.sum(oneh, axis=1, keepdims=True), cnt_ref.shape[1:])[None]


def _head_kernel(pooled_ref, cnt_ref, w1_ref, b1_ref, w2_ref, b2_ref,
                 w3_ref, b3_ref, o_ref):
    mean = pooled_ref[...] / jnp.maximum(cnt_ref[...], 1.0)
    h = jnp.concatenate([mean[0], mean[1]], axis=1)
    h = jnp.maximum(jnp.dot(h, w1_ref[...], preferred_element_type=jnp.float32)
                    + b1_ref[...], 0.0)
    h = jnp.maximum(jnp.dot(h, w2_ref[...], preferred_element_type=jnp.float32)
                    + b2_ref[...], 0.0)
    o_ref[...] = jnp.dot(h, w3_ref[...], preferred_element_type=jnp.float32) \
        + b3_ref[...]


def kernel(x1, edge_index1, batch1, x2, edge_index2, batch2,
           W1, b1, W2, b2, W3, b3,
           fc1_W, fc1_b, fc2_W, fc2_b, fc3_W, fc3_b):
    N, D = x1.shape
    H = W1.shape[1]
    E = edge_index1.shape[1]
    OUT = fc3_W.shape[1]
    G = 64
    NPAD = -(-N // (_NS * _CHUNK)) * (_NS * _CHUNK)
    RPT = NPAD // _NS
    NCHUNK = -(-E // (_NS * _CHUNK))
    NCHUNK += NCHUNK % 2
    EPC = _NS * NCHUNK * _CHUNK

    def prep_edges(ei, b):
        pad = EPC - E
        src = jnp.concatenate(
            [ei[0] + b * NPAD, jnp.full((pad,), b * NPAD + N, jnp.int32)])
        dst = jnp.concatenate([ei[1], jnp.full((pad,), N, jnp.int32)])
        return (src.reshape(_NS * NCHUNK, _CHUNK),
                dst.reshape(_NS * NCHUNK, _CHUNK))

    s1, d1 = prep_edges(edge_index1, 0)
    s2, d2 = prep_edges(edge_index2, 1)
    src_all = jnp.concatenate([s1, s2], axis=0)
    dst_all = jnp.concatenate([d1, d2], axis=0)

    def padrows(a):
        return jnp.concatenate(
            [a, jnp.zeros((NPAD - N, a.shape[1]), a.dtype)], axis=0)

    Xp = jnp.concatenate([padrows(x1), padrows(x2)], axis=0)
    batch2d = jnp.stack([
        jnp.concatenate([batch1, jnp.full((NPAD - N,), -1, jnp.int32)]),
        jnp.concatenate([batch2, jnp.full((NPAD - N,), -1, jnp.int32)]),
    ])
    zeros16 = jnp.zeros((RPT, 16), jnp.float32)
    ones16 = jnp.ones((_CHUNK, 16), jnp.float32)
    zerosH = jnp.zeros((RPT, H), jnp.float32)

    deg16 = pl.kernel(
        _deg_body,
        out_type=jax.ShapeDtypeStruct((_NC * NPAD, 16), jnp.float32),
        mesh=_sc_mesh(),
        scratch_types=[
            pltpu.VMEM((NCHUNK, _CHUNK), jnp.int32),
            pltpu.VMEM((_CHUNK, 16), jnp.float32),
            pltpu.VMEM_SHARED((NPAD, 16), jnp.float32),
        ],
    )(dst_all, zeros16, ones16)
    degp = deg16[:, :1]

    bm = 2048
    grid = (_NC * NPAD // bm,)
    mm_scale = pl.pallas_call(
        functools.partial(_mm_scale_kernel, n_valid=N, npad=NPAD, bm=bm),
        grid=grid,
        in_specs=[
            pl.BlockSpec((bm, 1), lambda i: (i, 0)),
            pl.BlockSpec((bm, D), lambda i: (i, 0)),
            pl.BlockSpec((D, H), lambda i: (0, 0)),
        ],
        out_specs=pl.BlockSpec((bm, H), lambda i: (i, 0)),
        out_shape=jax.ShapeDtypeStruct((_NC * NPAD, H), jnp.float32),
    )

    def mm_mid(degp, part, xwt, bvec, W):
        return pl.pallas_call(
            functools.partial(_mm_mid_kernel, n_valid=N, npad=NPAD, bm=bm),
            grid=grid,
            in_specs=[
                pl.BlockSpec((bm, 1), lambda i: (i, 0)),
                pl.BlockSpec((bm, H), lambda i: (i, 0)),
                pl.BlockSpec((bm, H), lambda i: (i, 0)),
                pl.BlockSpec((1, H), lambda i: (0, 0)),
                pl.BlockSpec((H, H), lambda i: (0, 0)),
            ],
            out_specs=pl.BlockSpec((bm, H), lambda i: (i, 0)),
            out_shape=jax.ShapeDtypeStruct((_NC * NPAD, H), jnp.float32),
        )(degp, part, xwt, bvec[None, :], W)

    def sc_layer(xwt):
        return pl.kernel(
            _gather_scatter_body,
            out_type=jax.ShapeDtypeStruct((_NC * NPAD, H), jnp.float32),
            mesh=_sc_mesh(),
            scratch_types=[
                pltpu.VMEM((NCHUNK, _CHUNK), jnp.int32),
                pltpu.VMEM((NCHUNK, _CHUNK), jnp.int32),
                pltpu.VMEM((_CHUNK, H), jnp.float32),
                pltpu.VMEM_SHARED((NPAD, H), jnp.float32),
            ],
        )(xwt, src_all, dst_all, zerosH)

    xwt1 = mm_scale(degp, Xp, W1)
    part1 = sc_layer(xwt1)
    xwt2 = mm_mid(degp, part1, xwt1, b1, W2)
    part2 = sc_layer(xwt2)
    xwt3 = mm_mid(degp, part2, xwt2, b2, W3)
    part3 = sc_layer(xwt3)

    bk = 2048
    pooled, cnt = pl.pallas_call(
        functools.partial(_pool_kernel, n_valid=N, npad=NPAD, bk=bk, g=G),
        grid=(_NC, NPAD // bk),
        in_specs=[
            pl.BlockSpec((bk, 1), lambda b, k: (b * (NPAD // bk) + k, 0)),
            pl.BlockSpec((bk, H), lambda b, k: (b * (NPAD // bk) + k, 0)),
            pl.BlockSpec((bk, H), lambda b, k: (b * (NPAD // bk) + k, 0)),
            pl.BlockSpec((1, H), lambda b, k: (0, 0)),
            pl.BlockSpec((1, bk), lambda b, k: (b, k)),
        ],
        out_specs=[
            pl.BlockSpec((1, G, H), lambda b, k: (b, 0, 0)),
            pl.BlockSpec((1, G, H), lambda b, k: (b, 0, 0)),
        ],
        out_shape=[
            jax.ShapeDtypeStruct((_NC, G, H), jnp.float32),
            jax.ShapeDtypeStruct((_NC, G, H), jnp.float32),
        ],
    )(degp, part3, xwt3, b3[None, :], batch2d)

    out = pl.pallas_call(
        _head_kernel,
        out_shape=jax.ShapeDtypeStruct((G, OUT), jnp.float32),
    )(pooled, cnt, fc1_W, fc1_b[None, :], fc2_W, fc2_b[None, :],
      fc3_W, fc3_b[None, :])
    return out


# speedup vs baseline: 7.2031x; 7.2031x over previous
import functools

import jax
import jax.numpy as jnp
from jax import lax
from jax.experimental import pallas as pl
from jax.experimental.pallas import tpu as pltpu
from jax.experimental.pallas import tpu_sc as plsc

_NC = 2
_NS = 16
_CHUNK = 128


def _sc_mesh():
    return plsc.VectorSubcoreMesh(
        core_axis_name="c", subcore_axis_name="s", num_cores=_NC, num_subcores=_NS
    )


_BLK = 8


def _deg_body(dst_hbm, zeros_hbm, ones_hbm, out_hbm, dstv, onesv, deg_sh,
              *, nchunk):
    c = lax.axis_index("c")
    s = lax.axis_index("s")
    tid = c * _NS + s
    rpt = deg_sh.shape[0] // _NS
    pltpu.sync_copy(ones_hbm, onesv)
    pltpu.sync_copy(zeros_hbm, deg_sh.at[pl.ds(s * rpt, rpt)])
    plsc.subcore_barrier()

    def outer(t, carry):
        pltpu.sync_copy(dst_hbm.at[pl.ds(tid * nchunk + t * _BLK, _BLK)], dstv)

        def body(g, carry):
            pltpu.sync_copy(onesv, deg_sh.at[dstv.at[g]], add=True)
            return carry

        return lax.fori_loop(0, _BLK, body, carry)

    lax.fori_loop(0, nchunk // _BLK, outer, 0)
    plsc.subcore_barrier()
    pltpu.sync_copy(deg_sh.at[pl.ds(s * rpt, rpt)], out_hbm.at[pl.ds(tid * rpt, rpt)])


def _gather_scatter_body(xwt_hbm, src_hbm, dst_hbm, zeros_hbm, out_hbm,
                         srcv, dstv, buf, acc_sh, *, nchunk):
    c = lax.axis_index("c")
    s = lax.axis_index("s")
    tid = c * _NS + s
    rpt = acc_sh.shape[0] // _NS
    pltpu.sync_copy(zeros_hbm, acc_sh.at[pl.ds(s * rpt, rpt)])
    plsc.subcore_barrier()

    def outer(t, carry):
        pltpu.sync_copy(src_hbm.at[pl.ds(tid * nchunk + t * _BLK, _BLK)], srcv)
        pltpu.sync_copy(dst_hbm.at[pl.ds(tid * nchunk + t * _BLK, _BLK)], dstv)

        def body(g, carry):
            pltpu.sync_copy(xwt_hbm.at[srcv.at[g]], buf)
            pltpu.sync_copy(buf, acc_sh.at[dstv.at[g]], add=True)
            return carry

        return lax.fori_loop(0, _BLK, body, carry)

    lax.fori_loop(0, nchunk // _BLK, outer, 0)
    plsc.subcore_barrier()
    pltpu.sync_copy(acc_sh.at[pl.ds(s * rpt, rpt)], out_hbm.at[pl.ds(tid * rpt, rpt)])


def _dinv_block(deg_ref, row0, bm, n_valid, npad):
    row = row0 + lax.broadcasted_iota(jnp.int32, (bm, 1), 0)
    valid = (row % npad) < n_valid
    return jnp.where(valid, lax.rsqrt(deg_ref[...] + 1.0), 0.0)


def _mm_scale_kernel(deg_ref, x_ref, w_ref, o_ref, *, n_valid, npad, bm):
    dinv = _dinv_block(deg_ref, pl.program_id(0) * bm, bm, n_valid, npad)
    o_ref[...] = dinv * jnp.dot(x_ref[...], w_ref[...],
                                preferred_element_type=jnp.float32)


def _mm_mid_kernel(deg_ref, part_ref, xwt_ref, b_ref, w_ref, o_ref,
                   *, n_valid, npad, bm):
    dinv = _dinv_block(deg_ref, pl.program_id(0) * bm, bm, n_valid, npad)
    h = jnp.maximum(dinv * (part_ref[...] + xwt_ref[...]) + b_ref[...], 0.0)
    o_ref[...] = dinv * jnp.dot(h, w_ref[...], preferred_element_type=jnp.float32)


def _pool_kernel(deg_ref, part_ref, xwt_ref, b_ref, batch_ref,
                 pooled_ref, cnt_ref, *, n_valid, npad, bk, g):
    k = pl.program_id(1)
    dinv = _dinv_block(deg_ref, k * bk, bk, n_valid, npad)
    o3 = dinv * (part_ref[...] + xwt_ref[...]) + b_ref[...]
    batv = batch_ref[0, 0, :]
    oneh = (lax.broadcasted_iota(jnp.int32, (g, bk), 0) == batv[None, :]).astype(
        jnp.float32)

    @pl.when(k == 0)
    def _():
        pooled_ref[...] = jnp.zeros_like(pooled_ref)
        cnt_ref[...] = jnp.zeros_like(cnt_ref)

    pooled_ref[...] += jnp.dot(oneh, o3, preferred_element_type=jnp.float32)[None]
    cnt_ref[...] += jnp.broadcast_to(
        jnp.sum(oneh, axis=1, keepdims=True), cnt_ref.shape[1:])[None]


def _head_kernel(pooled_ref, cnt_ref, w1_ref, b1_ref, w2_ref, b2_ref,
                 w3_ref, b3_ref, o_ref):
    mean = pooled_ref[...] / jnp.maximum(cnt_ref[...], 1.0)
    h = jnp.concatenate([mean[0], mean[1]], axis=1)
    h = jnp.maximum(jnp.dot(h, w1_ref[...], preferred_element_type=jnp.float32)
                    + b1_ref[...], 0.0)
    h = jnp.maximum(jnp.dot(h, w2_ref[...], preferred_element_type=jnp.float32)
                    + b2_ref[...], 0.0)
    o_ref[...] = jnp.dot(h, w3_ref[...], preferred_element_type=jnp.float32) \
        + b3_ref[...]


def kernel(x1, edge_index1, batch1, x2, edge_index2, batch2,
           W1, b1, W2, b2, W3, b3,
           fc1_W, fc1_b, fc2_W, fc2_b, fc3_W, fc3_b):
    N, D = x1.shape
    H = W1.shape[1]
    E = edge_index1.shape[1]
    OUT = fc3_W.shape[1]
    G = 64
    NPAD = -(-N // (_NS * _CHUNK)) * (_NS * _CHUNK)
    RPT = NPAD // _NS
    NCHUNK = -(-(-(-E // (_NS * _CHUNK))) // 8) * 8
    EPC = _NS * NCHUNK * _CHUNK

    def prep_edges(ei, b):
        pad = EPC - E
        src = jnp.concatenate(
            [ei[0] + b * NPAD, jnp.full((pad,), b * NPAD + N, jnp.int32)])
        dst = jnp.concatenate([ei[1], jnp.full((pad,), N, jnp.int32)])
        return (src.reshape(_NS * NCHUNK, _CHUNK),
                dst.reshape(_NS * NCHUNK, _CHUNK))

    s1, d1 = prep_edges(edge_index1, 0)
    s2, d2 = prep_edges(edge_index2, 1)
    src_all = jnp.concatenate([s1, s2], axis=0)
    dst_all = jnp.concatenate([d1, d2], axis=0)

    def padrows(a):
        return jnp.concatenate(
            [a, jnp.zeros((NPAD - N, a.shape[1]), a.dtype)], axis=0)

    Xp = jnp.concatenate([padrows(x1), padrows(x2)], axis=0)
    batch2d = jnp.stack([
        jnp.concatenate([batch1, jnp.full((NPAD - N,), -1, jnp.int32)]),
        jnp.concatenate([batch2, jnp.full((NPAD - N,), -1, jnp.int32)]),
    ])[:, None, :]
    onesH = jnp.ones((_CHUNK, H), jnp.float32)
    zerosH = jnp.zeros((RPT, H), jnp.float32)

    degw = pl.kernel(
        functools.partial(_deg_body, nchunk=NCHUNK),
        out_type=jax.ShapeDtypeStruct((_NC * NPAD, H), jnp.float32),
        mesh=_sc_mesh(),
        scratch_types=[
            pltpu.VMEM((_BLK, _CHUNK), jnp.int32),
            pltpu.VMEM((_CHUNK, H), jnp.float32),
            pltpu.VMEM_SHARED((NPAD, H), jnp.float32),
        ],
    )(dst_all, zerosH, onesH)
    degp = degw[:, :1]

    bm = 2048
    grid = (_NC * NPAD // bm,)
    mm_scale = pl.pallas_call(
        functools.partial(_mm_scale_kernel, n_valid=N, npad=NPAD, bm=bm),
        grid=grid,
        in_specs=[
            pl.BlockSpec((bm, 1), lambda i: (i, 0)),
            pl.BlockSpec((bm, D), lambda i: (i, 0)),
            pl.BlockSpec((D, H), lambda i: (0, 0)),
        ],
        out_specs=pl.BlockSpec((bm, H), lambda i: (i, 0)),
        out_shape=jax.ShapeDtypeStruct((_NC * NPAD, H), jnp.float32),
    )

    def mm_mid(degp, part, xwt, bvec, W):
        return pl.pallas_call(
            functools.partial(_mm_mid_kernel, n_valid=N, npad=NPAD, bm=bm),
            grid=grid,
            in_specs=[
                pl.BlockSpec((bm, 1), lambda i: (i, 0)),
                pl.BlockSpec((bm, H), lambda i: (i, 0)),
                pl.BlockSpec((bm, H), lambda i: (i, 0)),
                pl.BlockSpec((1, H), lambda i: (0, 0)),
                pl.BlockSpec((H, H), lambda i: (0, 0)),
            ],
            out_specs=pl.BlockSpec((bm, H), lambda i: (i, 0)),
            out_shape=jax.ShapeDtypeStruct((_NC * NPAD, H), jnp.float32),
        )(degp, part, xwt, bvec[None, :], W)

    def sc_layer(xwt):
        return pl.kernel(
            functools.partial(_gather_scatter_body, nchunk=NCHUNK),
            out_type=jax.ShapeDtypeStruct((_NC * NPAD, H), jnp.float32),
            mesh=_sc_mesh(),
            scratch_types=[
                pltpu.VMEM((_BLK, _CHUNK), jnp.int32),
                pltpu.VMEM((_BLK, _CHUNK), jnp.int32),
                pltpu.VMEM((_CHUNK, H), jnp.float32),
                pltpu.VMEM_SHARED((NPAD, H), jnp.float32),
            ],
        )(xwt, src_all, dst_all, zerosH)

    xwt1 = mm_scale(degp, Xp, W1)
    part1 = sc_layer(xwt1)
    xwt2 = mm_mid(degp, part1, xwt1, b1, W2)
    part2 = sc_layer(xwt2)
    xwt3 = mm_mid(degp, part2, xwt2, b2, W3)
    part3 = sc_layer(xwt3)

    bk = 2048
    pooled, cnt = pl.pallas_call(
        functools.partial(_pool_kernel, n_valid=N, npad=NPAD, bk=bk, g=G),
        grid=(_NC, NPAD // bk),
        in_specs=[
            pl.BlockSpec((bk, 1), lambda b, k: (b * (NPAD // bk) + k, 0)),
            pl.BlockSpec((bk, H), lambda b, k: (b * (NPAD // bk) + k, 0)),
            pl.BlockSpec((bk, H), lambda b, k: (b * (NPAD // bk) + k, 0)),
            pl.BlockSpec((1, H), lambda b, k: (0, 0)),
            pl.BlockSpec((1, 1, bk), lambda b, k: (b, 0, k)),
        ],
        out_specs=[
            pl.BlockSpec((1, G, H), lambda b, k: (b, 0, 0)),
            pl.BlockSpec((1, G, H), lambda b, k: (b, 0, 0)),
        ],
        out_shape=[
            jax.ShapeDtypeStruct((_NC, G, H), jnp.float32),
            jax.ShapeDtypeStruct((_NC, G, H), jnp.float32),
        ],
    )(degp, part3, xwt3, b3[None, :], batch2d)

    out = pl.pallas_call(
        _head_kernel,
        out_shape=jax.ShapeDtypeStruct((G, OUT), jnp.float32),
    )(pooled, cnt, fc1_W, fc1_b[None, :], fc2_W, fc2_b[None, :],
      fc3_W, fc3_b[None, :])
    return out


# speedup vs baseline: 8.2515x; 1.1456x over previous
import functools

import jax
import jax.numpy as jnp
from jax import lax
from jax.experimental import pallas as pl
from jax.experimental.pallas import tpu as pltpu
from jax.experimental.pallas import tpu_sc as plsc

_NC = 2
_NS = 16
_CHUNK = 128


def _sc_mesh():
    return plsc.VectorSubcoreMesh(
        core_axis_name="c", subcore_axis_name="s", num_cores=_NC, num_subcores=_NS
    )


_BLK = 8


def _deg_body(dst_hbm, zeros_hbm, ones_hbm, out_hbm, dstv, onesv, deg_sh,
              *, nchunk):
    c = lax.axis_index("c")
    s = lax.axis_index("s")
    tid = c * _NS + s
    rpt = deg_sh.shape[0] // _NS
    pltpu.sync_copy(ones_hbm, onesv)
    pltpu.sync_copy(zeros_hbm, deg_sh.at[pl.ds(s * rpt, rpt)])
    plsc.subcore_barrier()

    def outer(t, carry):
        pltpu.sync_copy(dst_hbm.at[pl.ds(tid * nchunk + t * _BLK, _BLK)], dstv)

        def body(g, carry):
            pltpu.sync_copy(onesv, deg_sh.at[dstv.at[g]], add=True)
            return carry

        return lax.fori_loop(0, _BLK, body, carry)

    lax.fori_loop(0, nchunk // _BLK, outer, 0)
    plsc.subcore_barrier()
    pltpu.sync_copy(deg_sh.at[pl.ds(s * rpt, rpt)], out_hbm.at[pl.ds(tid * rpt, rpt)])


def _gather_scatter_body(xwt_hbm, src_hbm, dst_hbm, zeros_hbm, out_hbm,
                         srcv, dstv, buf0, buf1, sem0, sem1, acc_sh, *, nchunk):
    c = lax.axis_index("c")
    s = lax.axis_index("s")
    tid = c * _NS + s
    rpt = acc_sh.shape[0] // _NS
    bufs = (buf0, buf1)
    sems = (sem0, sem1)
    pltpu.sync_copy(zeros_hbm, acc_sh.at[pl.ds(s * rpt, rpt)])
    plsc.subcore_barrier()

    def outer(t, carry):
        base = tid * nchunk + t * _BLK
        pltpu.sync_copy(src_hbm.at[pl.ds(base, _BLK)], srcv)
        pltpu.sync_copy(dst_hbm.at[pl.ds(base, _BLK)], dstv)
        descs = {0: pltpu.async_copy(xwt_hbm.at[srcv.at[0]], bufs[0], sems[0])}
        for g in range(_BLK):
            if g + 1 < _BLK:
                descs[g + 1] = pltpu.async_copy(
                    xwt_hbm.at[srcv.at[g + 1]], bufs[(g + 1) % 2],
                    sems[(g + 1) % 2])
            descs[g].wait()
            pltpu.sync_copy(bufs[g % 2], acc_sh.at[dstv.at[g]], add=True)
        return carry

    lax.fori_loop(0, nchunk // _BLK, outer, 0)
    plsc.subcore_barrier()
    pltpu.sync_copy(acc_sh.at[pl.ds(s * rpt, rpt)], out_hbm.at[pl.ds(tid * rpt, rpt)])


def _dinv_block(deg_ref, row0, bm, n_valid, npad):
    row = row0 + lax.broadcasted_iota(jnp.int32, (bm, 1), 0)
    valid = (row % npad) < n_valid
    return jnp.where(valid, lax.rsqrt(deg_ref[...] + 1.0), 0.0)


def _mm_scale_kernel(deg_ref, x_ref, w_ref, o_ref, *, n_valid, npad, bm):
    dinv = _dinv_block(deg_ref, pl.program_id(0) * bm, bm, n_valid, npad)
    o_ref[...] = dinv * jnp.dot(x_ref[...], w_ref[...],
                                preferred_element_type=jnp.float32)


def _mm_mid_kernel(deg_ref, part_ref, xwt_ref, b_ref, w_ref, o_ref,
                   *, n_valid, npad, bm):
    dinv = _dinv_block(deg_ref, pl.program_id(0) * bm, bm, n_valid, npad)
    h = jnp.maximum(dinv * (part_ref[...] + xwt_ref[...]) + b_ref[...], 0.0)
    o_ref[...] = dinv * jnp.dot(h, w_ref[...], preferred_element_type=jnp.float32)


def _pool_kernel(deg_ref, part_ref, xwt_ref, b_ref, batch_ref,
                 pooled_ref, cnt_ref, *, n_valid, npad, bk, g):
    k = pl.program_id(1)
    dinv = _dinv_block(deg_ref, k * bk, bk, n_valid, npad)
    o3 = dinv * (part_ref[...] + xwt_ref[...]) + b_ref[...]
    batv = batch_ref[0, 0, :]
    oneh = (lax.broadcasted_iota(jnp.int32, (g, bk), 0) == batv[None, :]).astype(
        jnp.float32)

    @pl.when(k == 0)
    def _():
        pooled_ref[...] = jnp.zeros_like(pooled_ref)
        cnt_ref[...] = jnp.zeros_like(cnt_ref)

    pooled_ref[...] += jnp.dot(oneh, o3, preferred_element_type=jnp.float32,
                               precision=lax.Precision.HIGHEST)[None]
    cnt_ref[...] += jnp.broadcast_to(
        jnp.sum(oneh, axis=1, keepdims=True), cnt_ref.shape[1:])[None]


def _head_kernel(pooled_ref, cnt_ref, w1_ref, b1_ref, w2_ref, b2_ref,
                 w3_ref, b3_ref, o_ref):
    mean = pooled_ref[...] / jnp.maximum(cnt_ref[...], 1.0)
    h = jnp.concatenate([mean[0], mean[1]], axis=1)
    h = jnp.maximum(jnp.dot(h, w1_ref[...], preferred_element_type=jnp.float32)
                    + b1_ref[...], 0.0)
    h = jnp.maximum(jnp.dot(h, w2_ref[...], preferred_element_type=jnp.float32)
                    + b2_ref[...], 0.0)
    o_ref[...] = jnp.dot(h, w3_ref[...], preferred_element_type=jnp.float32) \
        + b3_ref[...]


def kernel(x1, edge_index1, batch1, x2, edge_index2, batch2,
           W1, b1, W2, b2, W3, b3,
           fc1_W, fc1_b, fc2_W, fc2_b, fc3_W, fc3_b):
    N, D = x1.shape
    H = W1.shape[1]
    E = edge_index1.shape[1]
    OUT = fc3_W.shape[1]
    G = 64
    NPAD = -(-N // (_NS * _CHUNK)) * (_NS * _CHUNK)
    RPT = NPAD // _NS
    NCHUNK = -(-(-(-E // (_NS * _CHUNK))) // 8) * 8
    EPC = _NS * NCHUNK * _CHUNK

    def prep_edges(ei, b):
        pad = EPC - E
        src = jnp.concatenate(
            [ei[0] + b * NPAD, jnp.full((pad,), b * NPAD + N, jnp.int32)])
        dst = jnp.concatenate([ei[1], jnp.full((pad,), N, jnp.int32)])
        return (src.reshape(_NS * NCHUNK, _CHUNK),
                dst.reshape(_NS * NCHUNK, _CHUNK))

    s1, d1 = prep_edges(edge_index1, 0)
    s2, d2 = prep_edges(edge_index2, 1)
    src_all = jnp.concatenate([s1, s2], axis=0)
    dst_all = jnp.concatenate([d1, d2], axis=0)

    def padrows(a):
        return jnp.concatenate(
            [a, jnp.zeros((NPAD - N, a.shape[1]), a.dtype)], axis=0)

    Xp = jnp.concatenate([padrows(x1), padrows(x2)], axis=0)
    batch2d = jnp.stack([
        jnp.concatenate([batch1, jnp.full((NPAD - N,), -1, jnp.int32)]),
        jnp.concatenate([batch2, jnp.full((NPAD - N,), -1, jnp.int32)]),
    ])[:, None, :]
    onesH = jnp.ones((_CHUNK, H), jnp.float32)
    zerosH = jnp.zeros((RPT, H), jnp.float32)

    degw = pl.kernel(
        functools.partial(_deg_body, nchunk=NCHUNK),
        out_type=jax.ShapeDtypeStruct((_NC * NPAD, H), jnp.float32),
        mesh=_sc_mesh(),
        scratch_types=[
            pltpu.VMEM((_BLK, _CHUNK), jnp.int32),
            pltpu.VMEM((_CHUNK, H), jnp.float32),
            pltpu.VMEM_SHARED((NPAD, H), jnp.float32),
        ],
    )(dst_all, zerosH, onesH)
    degp = degw[:, :1]

    bm = 2048
    grid = (_NC * NPAD // bm,)
    mm_scale = pl.pallas_call(
        functools.partial(_mm_scale_kernel, n_valid=N, npad=NPAD, bm=bm),
        grid=grid,
        in_specs=[
            pl.BlockSpec((bm, 1), lambda i: (i, 0)),
            pl.BlockSpec((bm, D), lambda i: (i, 0)),
            pl.BlockSpec((D, H), lambda i: (0, 0)),
        ],
        out_specs=pl.BlockSpec((bm, H), lambda i: (i, 0)),
        out_shape=jax.ShapeDtypeStruct((_NC * NPAD, H), jnp.float32),
    )

    def mm_mid(degp, part, xwt, bvec, W):
        return pl.pallas_call(
            functools.partial(_mm_mid_kernel, n_valid=N, npad=NPAD, bm=bm),
            grid=grid,
            in_specs=[
                pl.BlockSpec((bm, 1), lambda i: (i, 0)),
                pl.BlockSpec((bm, H), lambda i: (i, 0)),
                pl.BlockSpec((bm, H), lambda i: (i, 0)),
                pl.BlockSpec((1, H), lambda i: (0, 0)),
                pl.BlockSpec((H, H), lambda i: (0, 0)),
            ],
            out_specs=pl.BlockSpec((bm, H), lambda i: (i, 0)),
            out_shape=jax.ShapeDtypeStruct((_NC * NPAD, H), jnp.float32),
        )(degp, part, xwt, bvec[None, :], W)

    def sc_layer(xwt):
        return pl.kernel(
            functools.partial(_gather_scatter_body, nchunk=NCHUNK),
            out_type=jax.ShapeDtypeStruct((_NC * NPAD, H), jnp.float32),
            mesh=_sc_mesh(),
            scratch_types=[
                pltpu.VMEM((_BLK, _CHUNK), jnp.int32),
                pltpu.VMEM((_BLK, _CHUNK), jnp.int32),
                pltpu.VMEM((_CHUNK, H), jnp.float32),
                pltpu.VMEM((_CHUNK, H), jnp.float32),
                pltpu.SemaphoreType.DMA,
                pltpu.SemaphoreType.DMA,
                pltpu.VMEM_SHARED((NPAD, H), jnp.float32),
            ],
        )(xwt, src_all, dst_all, zerosH)

    xwt1 = mm_scale(degp, Xp, W1)
    part1 = sc_layer(xwt1)
    xwt2 = mm_mid(degp, part1, xwt1, b1, W2)
    part2 = sc_layer(xwt2)
    xwt3 = mm_mid(degp, part2, xwt2, b2, W3)
    part3 = sc_layer(xwt3)

    bk = 2048
    pooled, cnt = pl.pallas_call(
        functools.partial(_pool_kernel, n_valid=N, npad=NPAD, bk=bk, g=G),
        grid=(_NC, NPAD // bk),
        in_specs=[
            pl.BlockSpec((bk, 1), lambda b, k: (b * (NPAD // bk) + k, 0)),
            pl.BlockSpec((bk, H), lambda b, k: (b * (NPAD // bk) + k, 0)),
            pl.BlockSpec((bk, H), lambda b, k: (b * (NPAD // bk) + k, 0)),
            pl.BlockSpec((1, H), lambda b, k: (0, 0)),
            pl.BlockSpec((1, 1, bk), lambda b, k: (b, 0, k)),
        ],
        out_specs=[
            pl.BlockSpec((1, G, H), lambda b, k: (b, 0, 0)),
            pl.BlockSpec((1, G, H), lambda b, k: (b, 0, 0)),
        ],
        out_shape=[
            jax.ShapeDtypeStruct((_NC, G, H), jnp.float32),
            jax.ShapeDtypeStruct((_NC, G, H), jnp.float32),
        ],
    )(degp, part3, xwt3, b3[None, :], batch2d)

    out = pl.pallas_call(
        _head_kernel,
        out_shape=jax.ShapeDtypeStruct((G, OUT), jnp.float32),
    )(pooled, cnt, fc1_W, fc1_b[None, :], fc2_W, fc2_b[None, :],
      fc3_W, fc3_b[None, :])
    return out
